# baseline (speedup 1.0000x reference)
import functools

import jax
import jax.numpy as jnp
from jax import lax
from jax.experimental import pallas as pl
from jax.experimental.pallas import tpu as pltpu
from jax.experimental.pallas import tpu_sc as plsc


B, S, D = 4, 8192, 1024
LANES = 16
NC, NS = 2, 16
NW = NC * NS
ROWS_PER_W = (B * S) // NW
R = 16

_mesh = plsc.VectorSubcoreMesh(core_axis_name="c", subcore_axis_name="s")


@functools.partial(
    pl.kernel,
    out_type=jax.ShapeDtypeStruct((B * S * D,), jnp.float32),
    mesh=_mesh,
    scratch_types=[
        pltpu.VMEM((R * D,), jnp.float32),
        pltpu.VMEM((R * D,), jnp.float32),
    ],
)
def _sc_add(x_hbm, te_hbm, o_hbm, x_v, te_v):
    wid = lax.axis_index("s") * NC + lax.axis_index("c")
    row0 = wid * ROWS_PER_W
    s0 = lax.rem(row0, S)

    def chunk(c, carry):
        rbase = (row0 + c * R) * D
        sbase = (s0 + c * R) * D
        pltpu.sync_copy(x_hbm.at[pl.ds(rbase, R * D)], x_v)
        pltpu.sync_copy(te_hbm.at[pl.ds(sbase, R * D)], te_v)

        def vec(i, carry2):
            sl = pl.ds(i * LANES, LANES)
            x_v[sl] = x_v[sl] + te_v[sl]
            return carry2

        lax.fori_loop(0, (R * D) // LANES, vec, None)
        pltpu.sync_copy(x_v, o_hbm.at[pl.ds(rbase, R * D)])
        return carry

    lax.fori_loop(0, ROWS_PER_W // R, chunk, None)


def kernel(x, time_embed):
    out = _sc_add(x.reshape(-1), time_embed.reshape(-1))
    return out.reshape(x.shape)


# speedup vs baseline: 1.3920x; 1.3920x over previous
import functools

import jax
import jax.numpy as jnp
from jax import lax
from jax.experimental import pallas as pl
from jax.experimental.pallas import tpu as pltpu
from jax.experimental.pallas import tpu_sc as plsc


B, S, D = 4, 8192, 1024
LANES = 16
NC, NS = 2, 16
NW = NC * NS
ROWS_PER_W = (B * S) // NW
R = 16

_mesh = plsc.VectorSubcoreMesh(core_axis_name="c", subcore_axis_name="s")


@functools.partial(
    pl.kernel,
    out_type=jax.ShapeDtypeStruct((B * S * D,), jnp.float32),
    mesh=_mesh,
    scratch_types=[
        pltpu.VMEM((R * D,), jnp.float32),
        pltpu.VMEM((R * D,), jnp.float32),
    ],
)
def _sc_add(x_hbm, te_hbm, o_hbm, x_v, te_v):
    wid = lax.axis_index("s") * NC + lax.axis_index("c")
    row0 = wid * ROWS_PER_W
    s0 = lax.rem(row0, S)

    def chunk(c, carry):
        rbase = (row0 + c * R) * D
        sbase = (s0 + c * R) * D
        pltpu.sync_copy(x_hbm.at[pl.ds(rbase, R * D)], x_v)
        pltpu.sync_copy(te_hbm.at[pl.ds(sbase, R * D)], te_v)

        @plsc.parallel_loop(0, R * D, step=LANES, unroll=8)
        def _add(i):
            sl = pl.ds(i, LANES)
            x_v[sl] = x_v[sl] + te_v[sl]
        pltpu.sync_copy(x_v, o_hbm.at[pl.ds(rbase, R * D)])
        return carry

    lax.fori_loop(0, ROWS_PER_W // R, chunk, None)


def kernel(x, time_embed):
    out = _sc_add(x.reshape(-1), time_embed.reshape(-1))
    return out.reshape(x.shape)


# speedup vs baseline: 8.5637x; 6.1521x over previous
import jax
import jax.numpy as jnp
from jax.experimental import pallas as pl


S_BLK = 512


def _add_pos_kernel(x_ref, pos_ref, o_ref):
    o_ref[...] = x_ref[...] + pos_ref[...][None, :, :]


def kernel(x, time_embed):
    B, S, d = x.shape
    grid = (S // S_BLK,)
    return pl.pallas_call(
        _add_pos_kernel,
        grid=grid,
        in_specs=[
            pl.BlockSpec((B, S_BLK, d), lambda i: (0, i, 0)),
            pl.BlockSpec((S_BLK, d), lambda i: (i, 0)),
        ],
        out_specs=pl.BlockSpec((B, S_BLK, d), lambda i: (0, i, 0)),
        out_shape=jax.ShapeDtypeStruct((B, S, d), x.dtype),
    )(x, time_embed)
